# baseline (speedup 1.0000x reference)
import functools

import jax
import jax.numpy as jnp
from jax import lax
from jax.experimental import pallas as pl
from jax.experimental.pallas import tpu as pltpu
from jax.experimental.pallas import tpu_sc as plsc

B, F, V, D, NUM, H, OUT = 16384, 26, 1000, 50, 64, 128, 2

def _tw_body(t_ref, w_ref, tw_ref):
    tw_ref[0] = jnp.dot(t_ref[0], w_ref[0], preferred_element_type=jnp.float32)


def _make_tw(tables, w1e):
    return pl.pallas_call(
        _tw_body,
        grid=(F,),
        in_specs=[
            pl.BlockSpec((1, V, D), lambda f: (f, 0, 0)),
            pl.BlockSpec((1, D, H), lambda f: (f, 0, 0)),
        ],
        out_specs=pl.BlockSpec((1, V, H), lambda f: (f, 0, 0)),
        out_shape=jax.ShapeDtypeStruct((F, V, H), jnp.float32),
    )(tables, w1e)


NC, NS = 2, 16
NW = NC * NS
BPW = B // NW
C = 16
IPC = C * F
CHUNKS = BPW // C
_SPLITS = (128, 128, 128, 32)


_PPER = 16 * F


def _sc_body(idx_hbm, tw_hbm, patt_hbm, out_hbm, idx_v, patt_v, rows0, rows1,
             acc0, acc1, gsem0, gsem1, osem0, osem1):
    wid = lax.axis_index("s") * NC + lax.axis_index("c")
    base_b = wid * BPW
    pltpu.sync_copy(idx_hbm.at[pl.ds(base_b * F, BPW * F)], idx_v)
    pltpu.sync_copy(patt_hbm.at[pl.ds(0, _PPER)], patt_v)

    def off_body(blk, carry0):
        base = blk * _PPER
        for j in range(_PPER // 16):
            s = pl.ds(base + j * 16, 16)
            idx_v[s] = idx_v[s] + patt_v[pl.ds(j * 16, 16)]
        return carry0

    lax.fori_loop(0, BPW * F // _PPER, off_body, 0, unroll=False)

    rows = (rows0, rows1)
    acc = (acc0, acc1)
    gsem = (gsem0, gsem1)
    osem = (osem0, osem1)

    def fire(ci, buf):
        off = 0
        for n in _SPLITS:
            pltpu.async_copy(
                tw_hbm.at[idx_v.at[pl.ds(ci * IPC + off, n)]],
                rows[buf].at[pl.ds(off, n)],
                gsem[buf],
            )
            off += n

    def drain_g(buf):
        pltpu.make_async_copy(tw_hbm.at[pl.ds(0, IPC)], rows[buf], gsem[buf]).wait()

    def drain_o(buf):
        pltpu.make_async_copy(acc[buf], out_hbm.at[pl.ds(0, C)], osem[buf]).wait()

    def compute(ci, buf):
        def b_body(bi, carry2):
            r0 = bi * F
            nh = H // 16
            accs = [rows[buf][r0, pl.ds(h * 16, 16)] for h in range(nh)]
            for f in range(1, F):
                for h in range(nh):
                    accs[h] = accs[h] + rows[buf][r0 + f, pl.ds(h * 16, 16)]
            for h in range(nh):
                acc[buf][bi, pl.ds(h * 16, 16)] = accs[h]
            return carry2

        lax.fori_loop(0, C, b_body, 0, unroll=False)
        pltpu.async_copy(acc[buf], out_hbm.at[pl.ds(base_b + ci * C, C)], osem[buf])

    fire(0, 0)
    K = CHUNKS // 2

    def body(k, carry):
        a = 2 * k
        fire(a + 1, 1)
        drain_g(0)

        @pl.when(k > 0)
        def _():
            drain_o(0)

        compute(a, 0)

        @pl.when(k < K - 1)
        def _():
            fire(a + 2, 0)

        drain_g(1)

        @pl.when(k > 0)
        def _():
            drain_o(1)

        compute(a + 1, 1)
        return carry

    lax.fori_loop(0, K, body, 0, unroll=False)
    drain_o(0)
    drain_o(1)


def _make_gsum(flat_idx, tw_flat, patt):
    mesh = plsc.VectorSubcoreMesh(core_axis_name="c", subcore_axis_name="s")
    f = functools.partial(
        pl.kernel,
        _sc_body,
        mesh=mesh,
        out_type=jax.ShapeDtypeStruct((B, H), jnp.float32),
        scratch_types=[
            pltpu.VMEM((BPW * F,), jnp.int32),
            pltpu.VMEM((_PPER,), jnp.int32),
            pltpu.VMEM((IPC, H), jnp.float32),
            pltpu.VMEM((IPC, H), jnp.float32),
            pltpu.VMEM((C, H), jnp.float32),
            pltpu.VMEM((C, H), jnp.float32),
            pltpu.SemaphoreType.DMA,
            pltpu.SemaphoreType.DMA,
            pltpu.SemaphoreType.DMA,
            pltpu.SemaphoreType.DMA,
        ],
    )()
    return f(flat_idx, tw_flat, patt)


_OBLK = 2048


def _out_body(h_ref, x_ref, w1n_ref, b1_ref, w2_ref, b2_ref, o_ref):
    z = (
        jnp.dot(x_ref[...], w1n_ref[...], preferred_element_type=jnp.float32)
        + b1_ref[...]
    )
    hrelu = jnp.maximum(h_ref[...] + z, 0.0)
    o_ref[...] = (
        jnp.dot(hrelu, w2_ref[...], preferred_element_type=jnp.float32)
        + b2_ref[...]
    )


def _make_out(gsum, x_num, w1n, b1, w2, b2):
    return pl.pallas_call(
        _out_body,
        grid=(B // _OBLK,),
        in_specs=[
            pl.BlockSpec((_OBLK, H), lambda i: (i, 0)),
            pl.BlockSpec((_OBLK, NUM), lambda i: (i, 0)),
            pl.BlockSpec((NUM, H), lambda i: (0, 0)),
            pl.BlockSpec((1, H), lambda i: (0, 0)),
            pl.BlockSpec((H, OUT), lambda i: (0, 0)),
            pl.BlockSpec((1, OUT), lambda i: (0, 0)),
        ],
        out_specs=pl.BlockSpec((_OBLK, OUT), lambda i: (i, 0)),
        out_shape=jax.ShapeDtypeStruct((B, OUT), jnp.float32),
    )(gsum, x_num, w1n, b1.reshape(1, H), w2, b2.reshape(1, OUT))


def kernel(x_cat, x_num, tables, W1, b1, W2, b2):
    w1e = W1[: F * D].reshape(F, D, H)
    w1n = W1[F * D :]
    tw = _make_tw(tables, w1e)

    patt = jnp.tile(jnp.arange(F, dtype=jnp.int32) * V, 16)
    gsum = _make_gsum(x_cat.astype(jnp.int32).reshape(-1), tw.reshape(F * V, H), patt)

    return _make_out(gsum, x_num, w1n, b1, W2, b2)


# speedup vs baseline: 1.0316x; 1.0316x over previous
import functools

import jax
import jax.numpy as jnp
from jax import lax
from jax.experimental import pallas as pl
from jax.experimental.pallas import tpu as pltpu
from jax.experimental.pallas import tpu_sc as plsc

B, F, V, D, NUM, H, OUT = 16384, 26, 1000, 50, 64, 128, 2

def _tw_body(t_ref, w_ref, tw_ref):
    tw_ref[0] = jnp.dot(t_ref[0], w_ref[0], preferred_element_type=jnp.float32)


def _make_tw(tables, w1e):
    return pl.pallas_call(
        _tw_body,
        grid=(F,),
        in_specs=[
            pl.BlockSpec((1, V, D), lambda f: (f, 0, 0)),
            pl.BlockSpec((1, D, H), lambda f: (f, 0, 0)),
        ],
        out_specs=pl.BlockSpec((1, V, H), lambda f: (f, 0, 0)),
        out_shape=jax.ShapeDtypeStruct((F, V, H), jnp.float32),
    )(tables, w1e)


NC, NS = 2, 16
NW = NC * NS
BPW = B // NW
C = 16
IPC = C * F
CHUNKS = BPW // C
_SPLITS = (128, 128, 128, 32)


def _sc_body(idx_hbm, tw_hbm, out_hbm, idx_v, rows0, rows1, acc0, acc1,
             gsem0, gsem1, osem0, osem1):
    wid = lax.axis_index("s") * NC + lax.axis_index("c")
    base_b = wid * BPW
    pltpu.sync_copy(idx_hbm.at[pl.ds(base_b * F, BPW * F)], idx_v)

    rows = (rows0, rows1)
    acc = (acc0, acc1)
    gsem = (gsem0, gsem1)
    osem = (osem0, osem1)

    def fire(ci, buf):
        off = 0
        for n in _SPLITS:
            pltpu.async_copy(
                tw_hbm.at[idx_v.at[pl.ds(ci * IPC + off, n)]],
                rows[buf].at[pl.ds(off, n)],
                gsem[buf],
            )
            off += n

    def drain_g(buf):
        pltpu.make_async_copy(tw_hbm.at[pl.ds(0, IPC)], rows[buf], gsem[buf]).wait()

    def drain_o(buf):
        pltpu.make_async_copy(acc[buf], out_hbm.at[pl.ds(0, C)], osem[buf]).wait()

    def compute(ci, buf):
        def b_body(bi, carry2):
            r0 = bi * F
            nh = H // 16
            accs = [rows[buf][r0, pl.ds(h * 16, 16)] for h in range(nh)]
            for f in range(1, F):
                for h in range(nh):
                    accs[h] = accs[h] + rows[buf][r0 + f, pl.ds(h * 16, 16)]
            for h in range(nh):
                acc[buf][bi, pl.ds(h * 16, 16)] = accs[h]
            return carry2

        lax.fori_loop(0, C, b_body, 0, unroll=False)
        pltpu.async_copy(acc[buf], out_hbm.at[pl.ds(base_b + ci * C, C)], osem[buf])

    fire(0, 0)
    K = CHUNKS // 2

    def body(k, carry):
        a = 2 * k
        fire(a + 1, 1)
        drain_g(0)

        @pl.when(k > 0)
        def _():
            drain_o(0)

        compute(a, 0)

        @pl.when(k < K - 1)
        def _():
            fire(a + 2, 0)

        drain_g(1)

        @pl.when(k > 0)
        def _():
            drain_o(1)

        compute(a + 1, 1)
        return carry

    lax.fori_loop(0, K, body, 0, unroll=False)
    drain_o(0)
    drain_o(1)


def _make_gsum(flat_idx, tw_flat):
    mesh = plsc.VectorSubcoreMesh(core_axis_name="c", subcore_axis_name="s")
    f = functools.partial(
        pl.kernel,
        _sc_body,
        mesh=mesh,
        out_type=jax.ShapeDtypeStruct((B, H), jnp.float32),
        scratch_types=[
            pltpu.VMEM((BPW * F,), jnp.int32),
            pltpu.VMEM((IPC, H), jnp.float32),
            pltpu.VMEM((IPC, H), jnp.float32),
            pltpu.VMEM((C, H), jnp.float32),
            pltpu.VMEM((C, H), jnp.float32),
            pltpu.SemaphoreType.DMA,
            pltpu.SemaphoreType.DMA,
            pltpu.SemaphoreType.DMA,
            pltpu.SemaphoreType.DMA,
        ],
    )()
    return f(flat_idx, tw_flat)


_OBLK = 2048


def _out_body(h_ref, x_ref, w1n_ref, b1_ref, w2_ref, b2_ref, o_ref):
    z = (
        jnp.dot(x_ref[...], w1n_ref[...], preferred_element_type=jnp.float32)
        + b1_ref[...]
    )
    hrelu = jnp.maximum(h_ref[...] + z, 0.0)
    o_ref[...] = (
        jnp.dot(hrelu, w2_ref[...], preferred_element_type=jnp.float32)
        + b2_ref[...]
    )


def _make_out(gsum, x_num, w1n, b1, w2, b2):
    return pl.pallas_call(
        _out_body,
        grid=(B // _OBLK,),
        in_specs=[
            pl.BlockSpec((_OBLK, H), lambda i: (i, 0)),
            pl.BlockSpec((_OBLK, NUM), lambda i: (i, 0)),
            pl.BlockSpec((NUM, H), lambda i: (0, 0)),
            pl.BlockSpec((1, H), lambda i: (0, 0)),
            pl.BlockSpec((H, OUT), lambda i: (0, 0)),
            pl.BlockSpec((1, OUT), lambda i: (0, 0)),
        ],
        out_specs=pl.BlockSpec((_OBLK, OUT), lambda i: (i, 0)),
        out_shape=jax.ShapeDtypeStruct((B, OUT), jnp.float32),
    )(gsum, x_num, w1n, b1.reshape(1, H), w2, b2.reshape(1, OUT))


def kernel(x_cat, x_num, tables, W1, b1, W2, b2):
    w1e = W1[: F * D].reshape(F, D, H)
    w1n = W1[F * D :]
    tw = _make_tw(tables, w1e)

    flat_idx = (x_cat.astype(jnp.int32) + jnp.arange(F, dtype=jnp.int32) * V).reshape(-1)
    gsum = _make_gsum(flat_idx, tw.reshape(F * V, H))

    return _make_out(gsum, x_num, w1n, b1, W2, b2)


# speedup vs baseline: 1.0522x; 1.0199x over previous
import functools

import jax
import jax.numpy as jnp
from jax import lax
from jax.experimental import pallas as pl
from jax.experimental.pallas import tpu as pltpu
from jax.experimental.pallas import tpu_sc as plsc

B, F, V, D, NUM, H, OUT = 16384, 26, 1000, 50, 64, 128, 2

def _tw_body(t_ref, w_ref, tw_ref):
    tw_ref[0] = jnp.dot(t_ref[0], w_ref[0], preferred_element_type=jnp.float32)


def _make_tw(tables, w1e):
    return pl.pallas_call(
        _tw_body,
        grid=(F,),
        in_specs=[
            pl.BlockSpec((1, V, D), lambda f: (f, 0, 0)),
            pl.BlockSpec((1, D, H), lambda f: (f, 0, 0)),
        ],
        out_specs=pl.BlockSpec((1, V, H), lambda f: (f, 0, 0)),
        out_shape=jax.ShapeDtypeStruct((F, V, H), jnp.float32),
    )(tables, w1e)


NC, NS = 2, 16
NW = NC * NS
BPW = B // NW
C = 8
IPC = C * F
CHUNKS = BPW // C
NBUF = 4
_SPLITS = (104, 104)


def _sc_body(idx_hbm, tw_hbm, out_hbm, idx_v, rows0, rows1, rows2, rows3,
             acc0, acc1, acc2, acc3, gsem0, gsem1, gsem2, gsem3,
             osem0, osem1, osem2, osem3):
    wid = lax.axis_index("s") * NC + lax.axis_index("c")
    base_b = wid * BPW
    pltpu.sync_copy(idx_hbm.at[pl.ds(base_b * F, BPW * F)], idx_v)

    rows = (rows0, rows1, rows2, rows3)
    acc = (acc0, acc1, acc2, acc3)
    gsem = (gsem0, gsem1, gsem2, gsem3)
    osem = (osem0, osem1, osem2, osem3)

    def fire(ci, buf):
        off = 0
        for n in _SPLITS:
            pltpu.async_copy(
                tw_hbm.at[idx_v.at[pl.ds(ci * IPC + off, n)]],
                rows[buf].at[pl.ds(off, n)],
                gsem[buf],
            )
            off += n

    def drain_g(buf):
        pltpu.make_async_copy(tw_hbm.at[pl.ds(0, IPC)], rows[buf], gsem[buf]).wait()

    def drain_o(buf):
        pltpu.make_async_copy(acc[buf], out_hbm.at[pl.ds(0, C)], osem[buf]).wait()

    def compute(ci, buf):
        def b_body(bi, carry2):
            r0 = bi * F
            nh = H // 16
            accs = [rows[buf][r0, pl.ds(h * 16, 16)] for h in range(nh)]
            for f in range(1, F):
                for h in range(nh):
                    accs[h] = accs[h] + rows[buf][r0 + f, pl.ds(h * 16, 16)]
            for h in range(nh):
                acc[buf][bi, pl.ds(h * 16, 16)] = accs[h]
            return carry2

        lax.fori_loop(0, C, b_body, 0, unroll=False)
        pltpu.async_copy(acc[buf], out_hbm.at[pl.ds(base_b + ci * C, C)], osem[buf])

    for b in range(NBUF - 1):
        fire(b, b)
    K = CHUNKS // NBUF

    def body(k, carry):
        for b in range(NBUF):
            ci = k * NBUF + b

            @pl.when(ci + NBUF - 1 < CHUNKS)
            def _():
                fire(ci + NBUF - 1, (b + NBUF - 1) % NBUF)

            drain_g(b)

            @pl.when(ci >= NBUF)
            def _():
                drain_o(b)

            compute(ci, b)
        return carry

    lax.fori_loop(0, K, body, 0, unroll=False)
    for b in range(NBUF):
        drain_o(b)


def _make_gsum(flat_idx, tw_flat):
    mesh = plsc.VectorSubcoreMesh(core_axis_name="c", subcore_axis_name="s")
    f = functools.partial(
        pl.kernel,
        _sc_body,
        mesh=mesh,
        out_type=jax.ShapeDtypeStruct((B, H), jnp.float32),
        scratch_types=(
            [pltpu.VMEM((BPW * F,), jnp.int32)]
            + [pltpu.VMEM((IPC, H), jnp.float32) for _ in range(NBUF)]
            + [pltpu.VMEM((C, H), jnp.float32) for _ in range(NBUF)]
            + [pltpu.SemaphoreType.DMA for _ in range(2 * NBUF)]
        ),
    )()
    return f(flat_idx, tw_flat)


_OBLK = 2048


def _out_body(h_ref, x_ref, w1n_ref, b1_ref, w2_ref, b2_ref, o_ref):
    z = (
        jnp.dot(x_ref[...], w1n_ref[...], preferred_element_type=jnp.float32)
        + b1_ref[...]
    )
    hrelu = jnp.maximum(h_ref[...] + z, 0.0)
    o_ref[...] = (
        jnp.dot(hrelu, w2_ref[...], preferred_element_type=jnp.float32)
        + b2_ref[...]
    )


def _make_out(gsum, x_num, w1n, b1, w2, b2):
    return pl.pallas_call(
        _out_body,
        grid=(B // _OBLK,),
        in_specs=[
            pl.BlockSpec((_OBLK, H), lambda i: (i, 0)),
            pl.BlockSpec((_OBLK, NUM), lambda i: (i, 0)),
            pl.BlockSpec((NUM, H), lambda i: (0, 0)),
            pl.BlockSpec((1, H), lambda i: (0, 0)),
            pl.BlockSpec((H, OUT), lambda i: (0, 0)),
            pl.BlockSpec((1, OUT), lambda i: (0, 0)),
        ],
        out_specs=pl.BlockSpec((_OBLK, OUT), lambda i: (i, 0)),
        out_shape=jax.ShapeDtypeStruct((B, OUT), jnp.float32),
    )(gsum, x_num, w1n, b1.reshape(1, H), w2, b2.reshape(1, OUT))


def kernel(x_cat, x_num, tables, W1, b1, W2, b2):
    w1e = W1[: F * D].reshape(F, D, H)
    w1n = W1[F * D :]
    tw = _make_tw(tables, w1e)

    flat_idx = (x_cat.astype(jnp.int32) + jnp.arange(F, dtype=jnp.int32) * V).reshape(-1)
    gsum = _make_gsum(flat_idx, tw.reshape(F * V, H))

    return _make_out(gsum, x_num, w1n, b1, W2, b2)
